# baseline (speedup 1.0000x reference)
import functools

import jax
import jax.numpy as jnp
from jax import lax
from jax.experimental import pallas as pl
from jax.experimental.pallas import tpu as pltpu
from jax.experimental.pallas import tpu_sc as plsc

NC = 2
NS = 16
K = 64
G = 8


def _remap_cols(cidx, q, base_node, nh):
    for j in range(K // 16):
        v = cidx[q, pl.ds(j * 16, 16)] - base_node
        ok = (v >= 0) & (v < nh)
        cidx[q, pl.ds(j * 16, 16)] = jnp.where(ok, v, nh)


def _sc_degree(col2d, zeros, ones_k, n_nodes, d):
    nb = col2d.shape[0] // NS
    ng = nb // G
    assert ng % 2 == 0
    nh = n_nodes // NC
    mesh = plsc.VectorSubcoreMesh(core_axis_name="c", subcore_axis_name="s")

    @functools.partial(
        pl.kernel,
        out_type=jax.ShapeDtypeStruct((NC, nh + 8, d), jnp.float32),
        mesh=mesh,
        scratch_types=[
            pltpu.VMEM((G, K), jnp.int32),
            pltpu.VMEM((G, K), jnp.int32),
            pltpu.VMEM((K, d), jnp.float32),
            pltpu.VMEM_SHARED((nh + 8, d), jnp.float32),
            pltpu.SemaphoreType.DMA,
            pltpu.SemaphoreType.DMA,
        ],
    )
    def k(col_h, z_h, ones_h, out_h, cidxA, cidxB, onev, acc, isem, ssem):
        c = lax.axis_index("c")
        s = lax.axis_index("s")
        base_node = c * nh
        base_row = s * nb

        @pl.when(s == 0)
        def _():
            pltpu.sync_copy(z_h, acc)

        pltpu.sync_copy(ones_h, onev)
        pltpu.async_copy(col_h.at[pl.ds(base_row, G)], cidxA, isem)
        plsc.subcore_barrier()

        def halfgroup(g, cidx, cidx_o):
            pltpu.make_async_copy(col_h.at[pl.ds(base_row, G)], cidx,
                                  isem).wait()
            for q in range(G):
                _remap_cols(cidx, q, base_node, nh)

            @pl.when(g + 1 < ng)
            def _():
                pltpu.async_copy(col_h.at[pl.ds(base_row + (g + 1) * G, G)],
                                 cidx_o, isem)

            for q in range(G):
                pltpu.async_copy(onev, acc.at[cidx.at[q]], ssem, add=True)
            for q in range(G):
                pltpu.make_async_copy(onev, acc.at[cidx.at[q]], ssem).wait()

        def body(m, carry):
            halfgroup(2 * m, cidxA, cidxB)
            halfgroup(2 * m + 1, cidxB, cidxA)
            return carry

        lax.fori_loop(0, ng // 2, body, 0)
        plsc.subcore_barrier()

        @pl.when(s == 0)
        def _():
            pltpu.sync_copy(acc, out_h.at[c])

    return k(col2d, zeros, ones_k)


def _sc_aggregate(hp, row2d, col2d, zeros, n_nodes, d):
    nb = row2d.shape[0] // NS
    ng = nb // G
    assert ng % 2 == 0
    nh = n_nodes // NC
    mesh = plsc.VectorSubcoreMesh(core_axis_name="c", subcore_axis_name="s")

    @functools.partial(
        pl.kernel,
        out_type=jax.ShapeDtypeStruct((NC, nh + 8, d), jnp.float32),
        mesh=mesh,
        scratch_types=[
            pltpu.VMEM((G, K), jnp.int32),
            pltpu.VMEM((G, K), jnp.int32),
            pltpu.VMEM((G, K), jnp.int32),
            pltpu.VMEM((G, K), jnp.int32),
            pltpu.VMEM((G, K, d), jnp.float32),
            pltpu.VMEM_SHARED((nh + 8, d), jnp.float32),
            pltpu.SemaphoreType.DMA((G,)),
            pltpu.SemaphoreType.DMA,
            pltpu.SemaphoreType.DMA,
        ],
    )
    def k(hp_h, row_h, col_h, z_h, out_h, ridxA, ridxB, cidxA, cidxB, rbuf,
          acc, gsem, isem, ssem):
        c = lax.axis_index("c")
        s = lax.axis_index("s")
        base_node = c * nh
        base_row = s * nb

        @pl.when(s == 0)
        def _():
            pltpu.sync_copy(z_h, acc)

        pltpu.async_copy(row_h.at[pl.ds(base_row, G)], ridxA, isem)
        pltpu.async_copy(col_h.at[pl.ds(base_row, G)], cidxA, isem)
        plsc.subcore_barrier()

        def halfgroup(g, ridx, cidx, ridx_o, cidx_o):
            pltpu.make_async_copy(row_h.at[pl.ds(base_row, G)], ridx,
                                  isem).wait()
            pltpu.make_async_copy(col_h.at[pl.ds(base_row, G)], cidx,
                                  isem).wait()
            for q in range(G):
                _remap_cols(cidx, q, base_node, nh)
            for q in range(G):
                pltpu.async_copy(hp_h.at[ridx.at[q]], rbuf.at[q], gsem.at[q])

            @pl.when(g + 1 < ng)
            def _():
                b2 = base_row + (g + 1) * G
                pltpu.async_copy(row_h.at[pl.ds(b2, G)], ridx_o, isem)
                pltpu.async_copy(col_h.at[pl.ds(b2, G)], cidx_o, isem)

            for q in range(G):
                pltpu.make_async_copy(hp_h.at[ridx.at[q]], rbuf.at[q],
                                      gsem.at[q]).wait()
                pltpu.async_copy(rbuf.at[q], acc.at[cidx.at[q]], ssem,
                                 add=True)
            for q in range(G):
                pltpu.make_async_copy(rbuf.at[q], acc.at[cidx.at[q]],
                                      ssem).wait()

        def body(m, carry):
            halfgroup(2 * m, ridxA, cidxA, ridxB, cidxB)
            halfgroup(2 * m + 1, ridxB, cidxB, ridxA, cidxA)
            return carry

        lax.fori_loop(0, ng // 2, body, 0)
        plsc.subcore_barrier()

        @pl.when(s == 0)
        def _():
            pltpu.sync_copy(acc, out_h.at[c])

    return k(hp, row2d, col2d, zeros)


def _knots(gref):
    return [gref[0, j] for j in range(12)]


def _kan_body(x, ts, bwT, swT_ref, scT):
    sig = 1.0 / (1.0 + jnp.exp(-x))
    out = jnp.dot(x * sig, bwT, preferred_element_type=jnp.float32)
    b = [jnp.where((x >= ts[j]) & (x < ts[j + 1]), 1.0, 0.0) for j in range(11)]
    for k in range(1, 4):
        b = [(x - ts[j]) / (ts[j + k] - ts[j]) * b[j]
             + (ts[j + k + 1] - x) / (ts[j + k + 1] - ts[j + 1]) * b[j + 1]
             for j in range(11 - k)]
    for j in range(8):
        out = out + jnp.dot(b[j], swT_ref[:, j, :] * scT,
                            preferred_element_type=jnp.float32)
    return out


def _dinv_block(dref):
    deg = dref[...][0][:, 0:1] + 1.0
    return 1.0 / jnp.sqrt(deg)


def _kan_prescale(x, degp, grow, bwT, swT, scT, blk):
    n, din = x.shape
    h = bwT.shape[1]
    bpp = (n // NC) // blk

    def body(xr, dr, gr, bwr, swr, scr, outr):
        ts = _knots(gr)
        outr[...] = _dinv_block(dr) * _kan_body(xr[...], ts, bwr[...], swr,
                                                scr[...])

    return pl.pallas_call(
        body,
        grid=(n // blk,),
        in_specs=[
            pl.BlockSpec((blk, din), lambda i: (i, 0)),
            pl.BlockSpec((1, blk, h), lambda i: (i // bpp, i % bpp, 0)),
            pl.BlockSpec(memory_space=pltpu.SMEM),
            pl.BlockSpec((din, h), lambda i: (0, 0)),
            pl.BlockSpec((din, 8, h), lambda i: (0, 0, 0)),
            pl.BlockSpec((din, h), lambda i: (0, 0)),
        ],
        out_specs=pl.BlockSpec((blk, h), lambda i: (i, 0)),
        out_shape=jax.ShapeDtypeStruct((n, h), jnp.float32),
    )(x, degp, grow, bwT, swT, scT)


def _combine_stats(parts, hp, degp, bias, blk):
    n, h = hp.shape
    bpp = (n // NC) // blk

    def body(pr, hr, dr, br, tr, sr, ssr):
        t = _dinv_block(dr) * (pr[...][0] + hr[...]) + br[...]
        tr[...] = t

        @pl.when(pl.program_id(0) == 0)
        def _():
            sr[...] = jnp.zeros_like(sr)
            ssr[...] = jnp.zeros_like(ssr)

        sr[...] += jnp.sum(t, axis=0, keepdims=True)
        ssr[...] += jnp.sum(t * t, axis=0, keepdims=True)

    return pl.pallas_call(
        body,
        grid=(n // blk,),
        in_specs=[
            pl.BlockSpec((1, blk, h), lambda i: (i // bpp, i % bpp, 0)),
            pl.BlockSpec((blk, h), lambda i: (i, 0)),
            pl.BlockSpec((1, blk, h), lambda i: (i // bpp, i % bpp, 0)),
            pl.BlockSpec((1, h), lambda i: (0, 0)),
        ],
        out_specs=[
            pl.BlockSpec((blk, h), lambda i: (i, 0)),
            pl.BlockSpec((1, h), lambda i: (0, 0)),
            pl.BlockSpec((1, h), lambda i: (0, 0)),
        ],
        out_shape=[
            jax.ShapeDtypeStruct((n, h), jnp.float32),
            jax.ShapeDtypeStruct((1, h), jnp.float32),
            jax.ShapeDtypeStruct((1, h), jnp.float32),
        ],
    )(parts, hp, degp, bias)


def _bn_block(t, sr, ssr, gammar, betar, n):
    mu = sr[...] / n
    var = ssr[...] / n - mu * mu
    return (t - mu) / jnp.sqrt(var + 1e-5) * gammar[...] + betar[...]


def _bn_kan_prescale(t, s, ss, gamma, beta, degp, grow, bwT, swT, scT, blk):
    n, h = t.shape
    hout = bwT.shape[1]
    bpp = (n // NC) // blk

    def body(tr, sr, ssr, gr_g, gr_b, dr, gr, bwr, swr, scr, bnr, hpr):
        bn = _bn_block(tr[...], sr, ssr, gr_g, gr_b, n)
        bnr[...] = bn
        ts = _knots(gr)
        hpr[...] = _dinv_block(dr) * _kan_body(bn, ts, bwr[...], swr, scr[...])

    return pl.pallas_call(
        body,
        grid=(n // blk,),
        in_specs=[
            pl.BlockSpec((blk, h), lambda i: (i, 0)),
            pl.BlockSpec((1, h), lambda i: (0, 0)),
            pl.BlockSpec((1, h), lambda i: (0, 0)),
            pl.BlockSpec((1, h), lambda i: (0, 0)),
            pl.BlockSpec((1, h), lambda i: (0, 0)),
            pl.BlockSpec((1, blk, h), lambda i: (i // bpp, i % bpp, 0)),
            pl.BlockSpec(memory_space=pltpu.SMEM),
            pl.BlockSpec((h, hout), lambda i: (0, 0)),
            pl.BlockSpec((h, 8, hout), lambda i: (0, 0, 0)),
            pl.BlockSpec((h, hout), lambda i: (0, 0)),
        ],
        out_specs=[
            pl.BlockSpec((blk, h), lambda i: (i, 0)),
            pl.BlockSpec((blk, hout), lambda i: (i, 0)),
        ],
        out_shape=[
            jax.ShapeDtypeStruct((n, h), jnp.float32),
            jax.ShapeDtypeStruct((n, hout), jnp.float32),
        ],
    )(t, s, ss, gamma, beta, degp, grow, bwT, swT, scT)


def _final(t1, s1, ss1, gamma, beta, x, bn0, grow, wparts, blk, ncls):
    n, h = t1.shape
    din = x.shape[1]
    (bw_x, sw_x, sc_x), (bw_a, sw_a, sc_a), (bw_b, sw_b, sc_b) = wparts

    def body(tr, sr, ssr, gr_g, gr_b, xr, bn0r, gr,
             bwxr, swxr, scxr, bwar, swar, scar, bwbr, swbr, scbr, outr):
        bn1 = _bn_block(tr[...], sr, ssr, gr_g, gr_b, n)
        ts = _knots(gr)
        z = (_kan_body(xr[...], ts, bwxr[...], swxr, scxr[...])
             + _kan_body(bn0r[...], ts, bwar[...], swar, scar[...])
             + _kan_body(bn1, ts, bwbr[...], swbr, scbr[...]))
        m = jnp.max(z, axis=1, keepdims=True)
        sh = z - m
        outr[...] = sh - jnp.log(jnp.sum(jnp.exp(sh), axis=1, keepdims=True))

    wspec = lambda a, b: pl.BlockSpec((a, b), lambda i: (0, 0))
    wspec3 = lambda a, b: pl.BlockSpec((a, 8, b), lambda i: (0, 0, 0))
    return pl.pallas_call(
        body,
        grid=(n // blk,),
        in_specs=[
            pl.BlockSpec((blk, h), lambda i: (i, 0)),
            pl.BlockSpec((1, h), lambda i: (0, 0)),
            pl.BlockSpec((1, h), lambda i: (0, 0)),
            pl.BlockSpec((1, h), lambda i: (0, 0)),
            pl.BlockSpec((1, h), lambda i: (0, 0)),
            pl.BlockSpec((blk, din), lambda i: (i, 0)),
            pl.BlockSpec((blk, h), lambda i: (i, 0)),
            pl.BlockSpec(memory_space=pltpu.SMEM),
            wspec(din, ncls), wspec3(din, ncls), wspec(din, ncls),
            wspec(h, ncls), wspec3(h, ncls), wspec(h, ncls),
            wspec(h, ncls), wspec3(h, ncls), wspec(h, ncls),
        ],
        out_specs=pl.BlockSpec((blk, ncls), lambda i: (i, 0)),
        out_shape=jax.ShapeDtypeStruct((n, ncls), jnp.float32),
    )(t1, s1, ss1, gamma, beta, x, bn0,
      grow, bw_x, sw_x, sc_x, bw_a, sw_a, sc_a, bw_b, sw_b, sc_b)


def kernel(x, edge_index, grid0, base_w0, spline_w0, scaler0, bias0, gamma0,
           beta0, grid1, base_w1, spline_w1, scaler1, bias1, gamma1, beta1,
           grid_out, base_w_out, spline_w_out, scaler_out):
    n, din = x.shape
    e = edge_index.shape[1]
    hid = base_w0.shape[0]
    ncls = base_w_out.shape[0]
    blk = 1000
    assert n % blk == 0 and (n // NC) % blk == 0 and n % NC == 0

    row = edge_index[0]
    col = edge_index[1]
    nh = n // NC
    chunk = NS * G * K
    e_pad = ((e + chunk - 1) // chunk) * chunk
    row2d = jnp.concatenate(
        [row, jnp.zeros((e_pad - e,), jnp.int32)]).reshape(-1, K)
    col2d = jnp.concatenate(
        [col, jnp.full((e_pad - e,), n, jnp.int32)]).reshape(-1, K)
    zeros_pad = jnp.zeros((nh + 8, hid), jnp.float32)
    ones_k = jnp.ones((K, hid), jnp.float32)

    bwT0 = base_w0.T
    swT0 = jnp.transpose(spline_w0, (1, 2, 0))
    scT0 = scaler0.T
    bwT1 = base_w1.T
    swT1 = jnp.transpose(spline_w1, (1, 2, 0))
    scT1 = scaler1.T
    bwTo = base_w_out.T
    swTo = jnp.transpose(spline_w_out, (1, 2, 0))
    scTo = scaler_out.T
    d0, d1 = din, din + hid
    wparts = (
        (bwTo[:d0], swTo[:d0], scTo[:d0]),
        (bwTo[d0:d1], swTo[d0:d1], scTo[d0:d1]),
        (bwTo[d1:], swTo[d1:], scTo[d1:]),
    )
    g0 = grid0[0:1]
    g1 = grid1[0:1]
    go = grid_out[0:1]

    degp = _sc_degree(col2d, zeros_pad, ones_k, n, hid)

    hp0 = _kan_prescale(x, degp, g0, bwT0, swT0, scT0, blk)
    p0 = _sc_aggregate(hp0, row2d, col2d, zeros_pad, n, hid)
    t0, s0, ss0 = _combine_stats(p0, hp0, degp, bias0[None, :], blk)

    bn0, hp1 = _bn_kan_prescale(t0, s0, ss0, gamma0[None, :], beta0[None, :],
                                degp, g1, bwT1, swT1, scT1, blk)
    p1 = _sc_aggregate(hp1, row2d, col2d, zeros_pad, n, hid)
    t1, s1, ss1 = _combine_stats(p1, hp1, degp, bias1[None, :], blk)

    return _final(t1, s1, ss1, gamma1[None, :], beta1[None, :], x, bn0,
                  go, wparts, blk, ncls)


# speedup vs baseline: 1.3383x; 1.3383x over previous
import functools

import jax
import jax.numpy as jnp
from jax import lax
from jax.experimental import pallas as pl
from jax.experimental.pallas import tpu as pltpu
from jax.experimental.pallas import tpu_sc as plsc

NC = 2
NS = 16
K = 128


def _remap_cols(cidx, base_node, nh):
    for j in range(K // 16):
        v = cidx[pl.ds(j * 16, 16)] - base_node
        ok = (v >= 0) & (v < nh)
        cidx[pl.ds(j * 16, 16)] = jnp.where(ok, v, nh)


def _sc_degree(col, zeros, ones_k, n_nodes, d):
    e = col.shape[0]
    ew = e // NS
    nb = ew // K
    nh = n_nodes // NC
    mesh = plsc.VectorSubcoreMesh(core_axis_name="c", subcore_axis_name="s")

    @functools.partial(
        pl.kernel,
        out_type=jax.ShapeDtypeStruct((NC, nh + 8, d), jnp.float32),
        mesh=mesh,
        scratch_types=[
            pltpu.VMEM((K,), jnp.int32),
            pltpu.VMEM((K, d), jnp.float32),
            pltpu.VMEM_SHARED((nh + 8, d), jnp.float32),
        ],
    )
    def k(col_h, z_h, ones_h, out_h, cidx, onev, acc):
        c = lax.axis_index("c")
        s = lax.axis_index("s")
        base_node = c * nh

        @pl.when(s == 0)
        def _():
            pltpu.sync_copy(z_h, acc)

        pltpu.sync_copy(ones_h, onev)
        plsc.subcore_barrier()

        def body(i, carry):
            b = s * ew + i * K
            pltpu.sync_copy(col_h.at[pl.ds(b, K)], cidx)
            _remap_cols(cidx, base_node, nh)
            pltpu.sync_copy(onev, acc.at[cidx], add=True)
            return carry

        lax.fori_loop(0, nb, body, 0)
        plsc.subcore_barrier()

        @pl.when(s == 0)
        def _():
            pltpu.sync_copy(acc, out_h.at[c])

    return k(col, zeros, ones_k)


def _sc_aggregate(hp, row, col, zeros, n_nodes, d):
    e = row.shape[0]
    ew = e // NS
    nb = ew // K
    nh = n_nodes // NC
    mesh = plsc.VectorSubcoreMesh(core_axis_name="c", subcore_axis_name="s")

    @functools.partial(
        pl.kernel,
        out_type=jax.ShapeDtypeStruct((NC, nh + 8, d), jnp.float32),
        mesh=mesh,
        scratch_types=[
            pltpu.VMEM((K,), jnp.int32),
            pltpu.VMEM((K,), jnp.int32),
            pltpu.VMEM((K, d), jnp.float32),
            pltpu.VMEM_SHARED((nh + 8, d), jnp.float32),
        ],
    )
    def k(hp_h, row_h, col_h, z_h, out_h, ridx, cidx, rbuf, acc):
        c = lax.axis_index("c")
        s = lax.axis_index("s")
        base_node = c * nh

        @pl.when(s == 0)
        def _():
            pltpu.sync_copy(z_h, acc)

        plsc.subcore_barrier()

        def body(i, carry):
            b = s * ew + i * K
            pltpu.sync_copy(row_h.at[pl.ds(b, K)], ridx)
            pltpu.sync_copy(col_h.at[pl.ds(b, K)], cidx)
            pltpu.sync_copy(hp_h.at[ridx], rbuf)
            _remap_cols(cidx, base_node, nh)
            pltpu.sync_copy(rbuf, acc.at[cidx], add=True)
            return carry

        lax.fori_loop(0, nb, body, 0)
        plsc.subcore_barrier()

        @pl.when(s == 0)
        def _():
            pltpu.sync_copy(acc, out_h.at[c])

    return k(hp, row, col, zeros)


def _knots(gref):
    return [gref[0, j] for j in range(12)]


def _kan_body(x, ts, bwT, swT_ref, scT):
    sig = 1.0 / (1.0 + jnp.exp(-x))
    out = jnp.dot(x * sig, bwT, preferred_element_type=jnp.float32)
    b = [jnp.where((x >= ts[j]) & (x < ts[j + 1]), 1.0, 0.0) for j in range(11)]
    for k in range(1, 4):
        b = [(x - ts[j]) / (ts[j + k] - ts[j]) * b[j]
             + (ts[j + k + 1] - x) / (ts[j + k + 1] - ts[j + 1]) * b[j + 1]
             for j in range(11 - k)]
    for j in range(8):
        out = out + jnp.dot(b[j], swT_ref[:, j, :] * scT,
                            preferred_element_type=jnp.float32)
    return out


def _dinv_block(dref):
    deg = dref[...][0][:, 0:1] + 1.0
    return 1.0 / jnp.sqrt(deg)


def _kan_prescale(x, degp, grow, bwT, swT, scT, blk):
    n, din = x.shape
    h = bwT.shape[1]
    bpp = (n // NC) // blk

    def body(xr, dr, gr, bwr, swr, scr, outr):
        ts = _knots(gr)
        outr[...] = _dinv_block(dr) * _kan_body(xr[...], ts, bwr[...], swr,
                                                scr[...])

    return pl.pallas_call(
        body,
        grid=(n // blk,),
        in_specs=[
            pl.BlockSpec((blk, din), lambda i: (i, 0)),
            pl.BlockSpec((1, blk, h), lambda i: (i // bpp, i % bpp, 0)),
            pl.BlockSpec(memory_space=pltpu.SMEM),
            pl.BlockSpec((din, h), lambda i: (0, 0)),
            pl.BlockSpec((din, 8, h), lambda i: (0, 0, 0)),
            pl.BlockSpec((din, h), lambda i: (0, 0)),
        ],
        out_specs=pl.BlockSpec((blk, h), lambda i: (i, 0)),
        out_shape=jax.ShapeDtypeStruct((n, h), jnp.float32),
    )(x, degp, grow, bwT, swT, scT)


def _combine_stats(parts, hp, degp, bias, blk):
    n, h = hp.shape
    bpp = (n // NC) // blk

    def body(pr, hr, dr, br, tr, sr, ssr):
        t = _dinv_block(dr) * (pr[...][0] + hr[...]) + br[...]
        tr[...] = t

        @pl.when(pl.program_id(0) == 0)
        def _():
            sr[...] = jnp.zeros_like(sr)
            ssr[...] = jnp.zeros_like(ssr)

        sr[...] += jnp.sum(t, axis=0, keepdims=True)
        ssr[...] += jnp.sum(t * t, axis=0, keepdims=True)

    return pl.pallas_call(
        body,
        grid=(n // blk,),
        in_specs=[
            pl.BlockSpec((1, blk, h), lambda i: (i // bpp, i % bpp, 0)),
            pl.BlockSpec((blk, h), lambda i: (i, 0)),
            pl.BlockSpec((1, blk, h), lambda i: (i // bpp, i % bpp, 0)),
            pl.BlockSpec((1, h), lambda i: (0, 0)),
        ],
        out_specs=[
            pl.BlockSpec((blk, h), lambda i: (i, 0)),
            pl.BlockSpec((1, h), lambda i: (0, 0)),
            pl.BlockSpec((1, h), lambda i: (0, 0)),
        ],
        out_shape=[
            jax.ShapeDtypeStruct((n, h), jnp.float32),
            jax.ShapeDtypeStruct((1, h), jnp.float32),
            jax.ShapeDtypeStruct((1, h), jnp.float32),
        ],
    )(parts, hp, degp, bias)


def _bn_block(t, sr, ssr, gammar, betar, n):
    mu = sr[...] / n
    var = ssr[...] / n - mu * mu
    return (t - mu) / jnp.sqrt(var + 1e-5) * gammar[...] + betar[...]


def _bn_kan_prescale(t, s, ss, gamma, beta, degp, grow, bwT, swT, scT, blk):
    n, h = t.shape
    hout = bwT.shape[1]
    bpp = (n // NC) // blk

    def body(tr, sr, ssr, gr_g, gr_b, dr, gr, bwr, swr, scr, bnr, hpr):
        bn = _bn_block(tr[...], sr, ssr, gr_g, gr_b, n)
        bnr[...] = bn
        ts = _knots(gr)
        hpr[...] = _dinv_block(dr) * _kan_body(bn, ts, bwr[...], swr, scr[...])

    return pl.pallas_call(
        body,
        grid=(n // blk,),
        in_specs=[
            pl.BlockSpec((blk, h), lambda i: (i, 0)),
            pl.BlockSpec((1, h), lambda i: (0, 0)),
            pl.BlockSpec((1, h), lambda i: (0, 0)),
            pl.BlockSpec((1, h), lambda i: (0, 0)),
            pl.BlockSpec((1, h), lambda i: (0, 0)),
            pl.BlockSpec((1, blk, h), lambda i: (i // bpp, i % bpp, 0)),
            pl.BlockSpec(memory_space=pltpu.SMEM),
            pl.BlockSpec((h, hout), lambda i: (0, 0)),
            pl.BlockSpec((h, 8, hout), lambda i: (0, 0, 0)),
            pl.BlockSpec((h, hout), lambda i: (0, 0)),
        ],
        out_specs=[
            pl.BlockSpec((blk, h), lambda i: (i, 0)),
            pl.BlockSpec((blk, hout), lambda i: (i, 0)),
        ],
        out_shape=[
            jax.ShapeDtypeStruct((n, h), jnp.float32),
            jax.ShapeDtypeStruct((n, hout), jnp.float32),
        ],
    )(t, s, ss, gamma, beta, degp, grow, bwT, swT, scT)


def _final(t1, s1, ss1, gamma, beta, x, bn0, grow, wparts, blk, ncls):
    n, h = t1.shape
    din = x.shape[1]
    (bw_x, sw_x, sc_x), (bw_a, sw_a, sc_a), (bw_b, sw_b, sc_b) = wparts

    def body(tr, sr, ssr, gr_g, gr_b, xr, bn0r, gr,
             bwxr, swxr, scxr, bwar, swar, scar, bwbr, swbr, scbr, outr):
        bn1 = _bn_block(tr[...], sr, ssr, gr_g, gr_b, n)
        ts = _knots(gr)
        z = (_kan_body(xr[...], ts, bwxr[...], swxr, scxr[...])
             + _kan_body(bn0r[...], ts, bwar[...], swar, scar[...])
             + _kan_body(bn1, ts, bwbr[...], swbr, scbr[...]))
        m = jnp.max(z, axis=1, keepdims=True)
        sh = z - m
        outr[...] = sh - jnp.log(jnp.sum(jnp.exp(sh), axis=1, keepdims=True))

    wspec = lambda a, b: pl.BlockSpec((a, b), lambda i: (0, 0))
    wspec3 = lambda a, b: pl.BlockSpec((a, 8, b), lambda i: (0, 0, 0))
    return pl.pallas_call(
        body,
        grid=(n // blk,),
        in_specs=[
            pl.BlockSpec((blk, h), lambda i: (i, 0)),
            pl.BlockSpec((1, h), lambda i: (0, 0)),
            pl.BlockSpec((1, h), lambda i: (0, 0)),
            pl.BlockSpec((1, h), lambda i: (0, 0)),
            pl.BlockSpec((1, h), lambda i: (0, 0)),
            pl.BlockSpec((blk, din), lambda i: (i, 0)),
            pl.BlockSpec((blk, h), lambda i: (i, 0)),
            pl.BlockSpec(memory_space=pltpu.SMEM),
            wspec(din, ncls), wspec3(din, ncls), wspec(din, ncls),
            wspec(h, ncls), wspec3(h, ncls), wspec(h, ncls),
            wspec(h, ncls), wspec3(h, ncls), wspec(h, ncls),
        ],
        out_specs=pl.BlockSpec((blk, ncls), lambda i: (i, 0)),
        out_shape=jax.ShapeDtypeStruct((n, ncls), jnp.float32),
    )(t1, s1, ss1, gamma, beta, x, bn0,
      grow, bw_x, sw_x, sc_x, bw_a, sw_a, sc_a, bw_b, sw_b, sc_b)


def kernel(x, edge_index, grid0, base_w0, spline_w0, scaler0, bias0, gamma0,
           beta0, grid1, base_w1, spline_w1, scaler1, bias1, gamma1, beta1,
           grid_out, base_w_out, spline_w_out, scaler_out):
    n, din = x.shape
    e = edge_index.shape[1]
    hid = base_w0.shape[0]
    ncls = base_w_out.shape[0]
    blk = 1000
    assert n % blk == 0 and (n // NC) % blk == 0 and n % NC == 0

    row = edge_index[0]
    col = edge_index[1]
    nh = n // NC
    chunk = NS * K
    e_pad = ((e + chunk - 1) // chunk) * chunk
    row_p = jnp.concatenate([row, jnp.zeros((e_pad - e,), jnp.int32)])
    col_p = jnp.concatenate([col, jnp.full((e_pad - e,), n, jnp.int32)])
    zeros_pad = jnp.zeros((nh + 8, hid), jnp.float32)
    ones_k = jnp.ones((K, hid), jnp.float32)

    bwT0 = base_w0.T
    swT0 = jnp.transpose(spline_w0, (1, 2, 0))
    scT0 = scaler0.T
    bwT1 = base_w1.T
    swT1 = jnp.transpose(spline_w1, (1, 2, 0))
    scT1 = scaler1.T
    bwTo = base_w_out.T
    swTo = jnp.transpose(spline_w_out, (1, 2, 0))
    scTo = scaler_out.T
    d0, d1 = din, din + hid
    wparts = (
        (bwTo[:d0], swTo[:d0], scTo[:d0]),
        (bwTo[d0:d1], swTo[d0:d1], scTo[d0:d1]),
        (bwTo[d1:], swTo[d1:], scTo[d1:]),
    )
    g0 = grid0[0:1]
    g1 = grid1[0:1]
    go = grid_out[0:1]

    degp = _sc_degree(col_p, zeros_pad, ones_k, n, hid)

    hp0 = _kan_prescale(x, degp, g0, bwT0, swT0, scT0, blk)
    p0 = _sc_aggregate(hp0, row_p, col_p, zeros_pad, n, hid)
    t0, s0, ss0 = _combine_stats(p0, hp0, degp, bias0[None, :], blk)

    bn0, hp1 = _bn_kan_prescale(t0, s0, ss0, gamma0[None, :], beta0[None, :],
                                degp, g1, bwT1, swT1, scT1, blk)
    p1 = _sc_aggregate(hp1, row_p, col_p, zeros_pad, n, hid)
    t1, s1, ss1 = _combine_stats(p1, hp1, degp, bias1[None, :], blk)

    return _final(t1, s1, ss1, gamma1[None, :], beta1[None, :], x, bn0,
                  go, wparts, blk, ncls)


# speedup vs baseline: 1.4175x; 1.0591x over previous
import functools

import jax
import jax.numpy as jnp
from jax import lax
from jax.experimental import pallas as pl
from jax.experimental.pallas import tpu as pltpu
from jax.experimental.pallas import tpu_sc as plsc

NC = 2
NS = 16
K = 128


def _remap_cols(cidx, base_node, nh):
    for j in range(K // 16):
        v = cidx[pl.ds(j * 16, 16)] - base_node
        ok = (v >= 0) & (v < nh)
        cidx[pl.ds(j * 16, 16)] = jnp.where(ok, v, nh)


def _sc_degree(col, zeros, ones_k, n_nodes, d):
    e = col.shape[0]
    ew = e // NS
    nb = ew // K
    nh = n_nodes // NC
    mesh = plsc.VectorSubcoreMesh(core_axis_name="c", subcore_axis_name="s")

    @functools.partial(
        pl.kernel,
        out_type=jax.ShapeDtypeStruct((NC, nh + 8, d), jnp.float32),
        mesh=mesh,
        scratch_types=[
            pltpu.VMEM((K,), jnp.int32),
            pltpu.VMEM((K, d), jnp.float32),
            pltpu.VMEM_SHARED((nh + 8, d), jnp.float32),
        ],
    )
    def k(col_h, z_h, ones_h, out_h, cidx, onev, acc):
        c = lax.axis_index("c")
        s = lax.axis_index("s")
        base_node = c * nh

        @pl.when(s == 0)
        def _():
            pltpu.sync_copy(z_h, acc)

        pltpu.sync_copy(ones_h, onev)
        plsc.subcore_barrier()

        def body(i, carry):
            b = s * ew + i * K
            pltpu.sync_copy(col_h.at[pl.ds(b, K)], cidx)
            _remap_cols(cidx, base_node, nh)
            pltpu.sync_copy(onev, acc.at[cidx], add=True)
            return carry

        lax.fori_loop(0, nb, body, 0)
        plsc.subcore_barrier()

        @pl.when(s == 0)
        def _():
            pltpu.sync_copy(acc, out_h.at[c])

    return k(col, zeros, ones_k)


def _sc_aggregate(hp, row, col, zeros, n_nodes, d):
    e = row.shape[0]
    ew = e // NS
    nb = ew // K
    nh = n_nodes // NC
    mesh = plsc.VectorSubcoreMesh(core_axis_name="c", subcore_axis_name="s")

    assert nb % 2 == 0

    @functools.partial(
        pl.kernel,
        out_type=jax.ShapeDtypeStruct((NC, nh + 8, d), jnp.float32),
        mesh=mesh,
        scratch_types=[
            pltpu.VMEM((K,), jnp.int32),
            pltpu.VMEM((K,), jnp.int32),
            pltpu.VMEM((K,), jnp.int32),
            pltpu.VMEM((K,), jnp.int32),
            pltpu.VMEM((K, d), jnp.float32),
            pltpu.VMEM((K, d), jnp.float32),
            pltpu.VMEM_SHARED((nh + 8, d), jnp.float32),
            pltpu.SemaphoreType.DMA,
            pltpu.SemaphoreType.DMA,
            pltpu.SemaphoreType.DMA,
        ],
    )
    def k(hp_h, row_h, col_h, z_h, out_h, ridxA, ridxB, cidxA, cidxB,
          rbufA, rbufB, acc, gsemA, gsemB, isem):
        c = lax.axis_index("c")
        s = lax.axis_index("s")
        base_node = c * nh
        b0 = s * ew

        @pl.when(s == 0)
        def _():
            pltpu.sync_copy(z_h, acc)

        pltpu.sync_copy(row_h.at[pl.ds(b0, K)], ridxA)
        pltpu.sync_copy(col_h.at[pl.ds(b0, K)], cidxA)
        pltpu.async_copy(hp_h.at[ridxA], rbufA, gsemA)
        pltpu.async_copy(row_h.at[pl.ds(b0 + K, K)], ridxB, isem)
        pltpu.async_copy(col_h.at[pl.ds(b0 + K, K)], cidxB, isem)
        plsc.subcore_barrier()

        def halfstep(i, ridx, cidx, rbuf, gsem, ridx_o, cidx_o, rbuf_o,
                     gsem_o):
            pltpu.make_async_copy(hp_h.at[ridx], rbuf, gsem).wait()

            @pl.when(i + 1 < nb)
            def _():
                pltpu.make_async_copy(row_h.at[pl.ds(b0, K)], ridx_o,
                                      isem).wait()
                pltpu.make_async_copy(col_h.at[pl.ds(b0, K)], cidx_o,
                                      isem).wait()
                pltpu.async_copy(hp_h.at[ridx_o], rbuf_o, gsem_o)

            _remap_cols(cidx, base_node, nh)
            pltpu.sync_copy(rbuf, acc.at[cidx], add=True)

            @pl.when(i + 2 < nb)
            def _():
                b2 = b0 + (i + 2) * K
                pltpu.async_copy(row_h.at[pl.ds(b2, K)], ridx, isem)
                pltpu.async_copy(col_h.at[pl.ds(b2, K)], cidx, isem)

        def body(m, carry):
            halfstep(2 * m, ridxA, cidxA, rbufA, gsemA,
                     ridxB, cidxB, rbufB, gsemB)
            halfstep(2 * m + 1, ridxB, cidxB, rbufB, gsemB,
                     ridxA, cidxA, rbufA, gsemA)
            return carry

        lax.fori_loop(0, nb // 2, body, 0)
        plsc.subcore_barrier()

        @pl.when(s == 0)
        def _():
            pltpu.sync_copy(acc, out_h.at[c])

    return k(hp, row, col, zeros)


def _knots(gref):
    return [gref[0, j] for j in range(12)]


def _kan_body(x, ts, bwT, swT_ref, scT):
    sig = 1.0 / (1.0 + jnp.exp(-x))
    out = jnp.dot(x * sig, bwT, preferred_element_type=jnp.float32)
    b = [jnp.where((x >= ts[j]) & (x < ts[j + 1]), 1.0, 0.0) for j in range(11)]
    for k in range(1, 4):
        b = [(x - ts[j]) / (ts[j + k] - ts[j]) * b[j]
             + (ts[j + k + 1] - x) / (ts[j + k + 1] - ts[j + 1]) * b[j + 1]
             for j in range(11 - k)]
    for j in range(8):
        out = out + jnp.dot(b[j], swT_ref[:, j, :] * scT,
                            preferred_element_type=jnp.float32)
    return out


def _dinv_block(dref):
    deg = dref[...][0][:, 0:1] + 1.0
    return 1.0 / jnp.sqrt(deg)


def _kan_prescale(x, degp, grow, bwT, swT, scT, blk):
    n, din = x.shape
    h = bwT.shape[1]
    bpp = (n // NC) // blk

    def body(xr, dr, gr, bwr, swr, scr, outr):
        ts = _knots(gr)
        outr[...] = _dinv_block(dr) * _kan_body(xr[...], ts, bwr[...], swr,
                                                scr[...])

    return pl.pallas_call(
        body,
        grid=(n // blk,),
        in_specs=[
            pl.BlockSpec((blk, din), lambda i: (i, 0)),
            pl.BlockSpec((1, blk, h), lambda i: (i // bpp, i % bpp, 0)),
            pl.BlockSpec(memory_space=pltpu.SMEM),
            pl.BlockSpec((din, h), lambda i: (0, 0)),
            pl.BlockSpec((din, 8, h), lambda i: (0, 0, 0)),
            pl.BlockSpec((din, h), lambda i: (0, 0)),
        ],
        out_specs=pl.BlockSpec((blk, h), lambda i: (i, 0)),
        out_shape=jax.ShapeDtypeStruct((n, h), jnp.float32),
    )(x, degp, grow, bwT, swT, scT)


def _combine_stats(parts, hp, degp, bias, blk):
    n, h = hp.shape
    bpp = (n // NC) // blk

    def body(pr, hr, dr, br, tr, sr, ssr):
        t = _dinv_block(dr) * (pr[...][0] + hr[...]) + br[...]
        tr[...] = t

        @pl.when(pl.program_id(0) == 0)
        def _():
            sr[...] = jnp.zeros_like(sr)
            ssr[...] = jnp.zeros_like(ssr)

        sr[...] += jnp.sum(t, axis=0, keepdims=True)
        ssr[...] += jnp.sum(t * t, axis=0, keepdims=True)

    return pl.pallas_call(
        body,
        grid=(n // blk,),
        in_specs=[
            pl.BlockSpec((1, blk, h), lambda i: (i // bpp, i % bpp, 0)),
            pl.BlockSpec((blk, h), lambda i: (i, 0)),
            pl.BlockSpec((1, blk, h), lambda i: (i // bpp, i % bpp, 0)),
            pl.BlockSpec((1, h), lambda i: (0, 0)),
        ],
        out_specs=[
            pl.BlockSpec((blk, h), lambda i: (i, 0)),
            pl.BlockSpec((1, h), lambda i: (0, 0)),
            pl.BlockSpec((1, h), lambda i: (0, 0)),
        ],
        out_shape=[
            jax.ShapeDtypeStruct((n, h), jnp.float32),
            jax.ShapeDtypeStruct((1, h), jnp.float32),
            jax.ShapeDtypeStruct((1, h), jnp.float32),
        ],
    )(parts, hp, degp, bias)


def _bn_block(t, sr, ssr, gammar, betar, n):
    mu = sr[...] / n
    var = ssr[...] / n - mu * mu
    return (t - mu) / jnp.sqrt(var + 1e-5) * gammar[...] + betar[...]


def _bn_kan_prescale(t, s, ss, gamma, beta, degp, grow, bwT, swT, scT, blk):
    n, h = t.shape
    hout = bwT.shape[1]
    bpp = (n // NC) // blk

    def body(tr, sr, ssr, gr_g, gr_b, dr, gr, bwr, swr, scr, bnr, hpr):
        bn = _bn_block(tr[...], sr, ssr, gr_g, gr_b, n)
        bnr[...] = bn
        ts = _knots(gr)
        hpr[...] = _dinv_block(dr) * _kan_body(bn, ts, bwr[...], swr, scr[...])

    return pl.pallas_call(
        body,
        grid=(n // blk,),
        in_specs=[
            pl.BlockSpec((blk, h), lambda i: (i, 0)),
            pl.BlockSpec((1, h), lambda i: (0, 0)),
            pl.BlockSpec((1, h), lambda i: (0, 0)),
            pl.BlockSpec((1, h), lambda i: (0, 0)),
            pl.BlockSpec((1, h), lambda i: (0, 0)),
            pl.BlockSpec((1, blk, h), lambda i: (i // bpp, i % bpp, 0)),
            pl.BlockSpec(memory_space=pltpu.SMEM),
            pl.BlockSpec((h, hout), lambda i: (0, 0)),
            pl.BlockSpec((h, 8, hout), lambda i: (0, 0, 0)),
            pl.BlockSpec((h, hout), lambda i: (0, 0)),
        ],
        out_specs=[
            pl.BlockSpec((blk, h), lambda i: (i, 0)),
            pl.BlockSpec((blk, hout), lambda i: (i, 0)),
        ],
        out_shape=[
            jax.ShapeDtypeStruct((n, h), jnp.float32),
            jax.ShapeDtypeStruct((n, hout), jnp.float32),
        ],
    )(t, s, ss, gamma, beta, degp, grow, bwT, swT, scT)


def _final(t1, s1, ss1, gamma, beta, x, bn0, grow, wparts, blk, ncls):
    n, h = t1.shape
    din = x.shape[1]
    (bw_x, sw_x, sc_x), (bw_a, sw_a, sc_a), (bw_b, sw_b, sc_b) = wparts

    def body(tr, sr, ssr, gr_g, gr_b, xr, bn0r, gr,
             bwxr, swxr, scxr, bwar, swar, scar, bwbr, swbr, scbr, outr):
        bn1 = _bn_block(tr[...], sr, ssr, gr_g, gr_b, n)
        ts = _knots(gr)
        z = (_kan_body(xr[...], ts, bwxr[...], swxr, scxr[...])
             + _kan_body(bn0r[...], ts, bwar[...], swar, scar[...])
             + _kan_body(bn1, ts, bwbr[...], swbr, scbr[...]))
        m = jnp.max(z, axis=1, keepdims=True)
        sh = z - m
        outr[...] = sh - jnp.log(jnp.sum(jnp.exp(sh), axis=1, keepdims=True))

    wspec = lambda a, b: pl.BlockSpec((a, b), lambda i: (0, 0))
    wspec3 = lambda a, b: pl.BlockSpec((a, 8, b), lambda i: (0, 0, 0))
    return pl.pallas_call(
        body,
        grid=(n // blk,),
        in_specs=[
            pl.BlockSpec((blk, h), lambda i: (i, 0)),
            pl.BlockSpec((1, h), lambda i: (0, 0)),
            pl.BlockSpec((1, h), lambda i: (0, 0)),
            pl.BlockSpec((1, h), lambda i: (0, 0)),
            pl.BlockSpec((1, h), lambda i: (0, 0)),
            pl.BlockSpec((blk, din), lambda i: (i, 0)),
            pl.BlockSpec((blk, h), lambda i: (i, 0)),
            pl.BlockSpec(memory_space=pltpu.SMEM),
            wspec(din, ncls), wspec3(din, ncls), wspec(din, ncls),
            wspec(h, ncls), wspec3(h, ncls), wspec(h, ncls),
            wspec(h, ncls), wspec3(h, ncls), wspec(h, ncls),
        ],
        out_specs=pl.BlockSpec((blk, ncls), lambda i: (i, 0)),
        out_shape=jax.ShapeDtypeStruct((n, ncls), jnp.float32),
    )(t1, s1, ss1, gamma, beta, x, bn0,
      grow, bw_x, sw_x, sc_x, bw_a, sw_a, sc_a, bw_b, sw_b, sc_b)


def kernel(x, edge_index, grid0, base_w0, spline_w0, scaler0, bias0, gamma0,
           beta0, grid1, base_w1, spline_w1, scaler1, bias1, gamma1, beta1,
           grid_out, base_w_out, spline_w_out, scaler_out):
    n, din = x.shape
    e = edge_index.shape[1]
    hid = base_w0.shape[0]
    ncls = base_w_out.shape[0]
    blk = 1000
    assert n % blk == 0 and (n // NC) % blk == 0 and n % NC == 0

    row = edge_index[0]
    col = edge_index[1]
    nh = n // NC
    chunk = NS * K * 2
    e_pad = ((e + chunk - 1) // chunk) * chunk
    row_p = jnp.concatenate([row, jnp.zeros((e_pad - e,), jnp.int32)])
    col_p = jnp.concatenate([col, jnp.full((e_pad - e,), n, jnp.int32)])
    zeros_pad = jnp.zeros((nh + 8, hid), jnp.float32)
    ones_k = jnp.ones((K, hid), jnp.float32)

    bwT0 = base_w0.T
    swT0 = jnp.transpose(spline_w0, (1, 2, 0))
    scT0 = scaler0.T
    bwT1 = base_w1.T
    swT1 = jnp.transpose(spline_w1, (1, 2, 0))
    scT1 = scaler1.T
    bwTo = base_w_out.T
    swTo = jnp.transpose(spline_w_out, (1, 2, 0))
    scTo = scaler_out.T
    d0, d1 = din, din + hid
    wparts = (
        (bwTo[:d0], swTo[:d0], scTo[:d0]),
        (bwTo[d0:d1], swTo[d0:d1], scTo[d0:d1]),
        (bwTo[d1:], swTo[d1:], scTo[d1:]),
    )
    g0 = grid0[0:1]
    g1 = grid1[0:1]
    go = grid_out[0:1]

    degp = _sc_degree(col_p, zeros_pad, ones_k, n, hid)

    hp0 = _kan_prescale(x, degp, g0, bwT0, swT0, scT0, blk)
    p0 = _sc_aggregate(hp0, row_p, col_p, zeros_pad, n, hid)
    t0, s0, ss0 = _combine_stats(p0, hp0, degp, bias0[None, :], blk)

    bn0, hp1 = _bn_kan_prescale(t0, s0, ss0, gamma0[None, :], beta0[None, :],
                                degp, g1, bwT1, swT1, scT1, blk)
    p1 = _sc_aggregate(hp1, row_p, col_p, zeros_pad, n, hid)
    t1, s1, ss1 = _combine_stats(p1, hp1, degp, bias1[None, :], blk)

    return _final(t1, s1, ss1, gamma1[None, :], beta1[None, :], x, bn0,
                  go, wparts, blk, ncls)
